# baseline (speedup 1.0000x reference)
import functools

import jax
import jax.numpy as jnp
from jax import lax
from jax.experimental import pallas as pl
from jax.experimental.pallas import tpu as pltpu
from jax.experimental.pallas import tpu_sc as plsc

DIM = 32
BATCH = 16384
L = 16
NC, NS = 2, 16
NW = NC * NS
B2 = 2 * BATCH
BPW = B2 // NW
TPC = 4
CHUNKS = BPW // TPC
SLOTS = 3 * TPC


def _body(tri_hbm, ent_hbm, rel_hbm, out_hbm, tri_v, blk, out_v, sems):
    wid = lax.axis_index("s") * NC + lax.axis_index("c")
    base = wid * BPW
    pltpu.sync_copy(tri_hbm.at[pl.ds(base * 3, BPW * 3)], tri_v)

    iota = lax.iota(jnp.int32, L)

    def chunk_idx(g):
        pos = jnp.minimum(g * SLOTS + iota, jnp.int32(BPW * 3 - 1))
        return plsc.load_gather(tri_v, [pos])

    def fire(g):
        idx = chunk_idx(g)
        p = g & 1
        sem = sems.at[p]
        for k in range(SLOTS):
            src = rel_hbm if k % 3 == 1 else ent_hbm
            start = pl.multiple_of(
                lax.shift_right_logical(idx[k], 7) * 128, 128)
            dst = pl.ds((p * SLOTS + k) * DIM, DIM)
            pltpu.make_async_copy(
                src.at[:, pl.ds(start, 128)], blk.at[dst], sem).start()

    def drain(g):
        sem = sems.at[g & 1]
        for _ in range(SLOTS):
            pltpu.make_async_copy(
                ent_hbm.at[:, pl.ds(0, 128)], blk.at[pl.ds(0, DIM)],
                sem).wait()

    def compute(g, acc):
        idx = chunk_idx(g)
        p = g & 1
        for j in range(TPC):
            s0 = (p * SLOTS + 3 * j) * DIM
            ch = jnp.full((L,), 0, jnp.int32) + (idx[3 * j] & 127)
            cr = jnp.full((L,), 0, jnp.int32) + (idx[3 * j + 1] & 127)
            ct = jnp.full((L,), 0, jnp.int32) + (idx[3 * j + 2] & 127)
            c = jnp.zeros((L,), jnp.float32)
            for half in range(2):
                rows = s0 + half * L + iota
                hv = plsc.load_gather(blk, [rows, ch])
                rv = plsc.load_gather(blk, [rows + DIM, cr])
                tv = plsc.load_gather(blk, [rows + 2 * DIM, ct])
                u = hv + rv - tv
                c = c + u * u
            s = jnp.sum(c)
            acc = jnp.where(iota == ((g * TPC + j) & 15), s, acc)

        @pl.when((g & 3) == 3)
        def _():
            am = jnp.maximum(acc, jnp.float32(1e-30))
            yi = jnp.int32(0x5F3759DF) - lax.shift_right_logical(
                plsc.bitcast(am, jnp.int32), 1)
            y = plsc.bitcast(yi, jnp.float32)
            for _ in range(3):
                y = y * (jnp.float32(1.5) - jnp.float32(0.5) * am * y * y)
            out_v[pl.ds((lax.shift_right_logical(g, 2)) * L, L)] = am * y
        return acc

    fire(0)

    def step(g, acc):
        @pl.when(g + 1 < CHUNKS)
        def _():
            fire(g + 1)
        drain(g)
        return compute(g, acc)
    lax.fori_loop(0, CHUNKS, step, jnp.zeros((L,), jnp.float32))

    pltpu.sync_copy(out_v, out_hbm.at[pl.ds(base, BPW)])


_transe_sc = functools.partial(
    pl.kernel,
    mesh=plsc.VectorSubcoreMesh(core_axis_name="c", subcore_axis_name="s"),
    compiler_params=pltpu.CompilerParams(needs_layout_passes=False),
    out_type=jax.ShapeDtypeStruct((B2,), jnp.float32),
    scratch_types=[
        pltpu.VMEM((BPW * 3,), jnp.int32),
        pltpu.VMEM((2 * SLOTS * DIM, 128), jnp.float32),
        pltpu.VMEM((BPW,), jnp.float32),
        pltpu.SemaphoreType.DMA((2,)),
    ],
)(_body)


def kernel(pos, neg, entity_W, relation_W):
    tri = jnp.concatenate([pos, neg], axis=0).reshape(-1)
    out = _transe_sc(tri, entity_W.T, relation_W.T)
    return out[:BATCH], out[BATCH:]
